# baseline (speedup 1.0000x reference)
import functools

import jax
import jax.numpy as jnp
from jax import lax
from jax.experimental import pallas as pl
from jax.experimental.pallas import tpu as pltpu
from jax.experimental.pallas import tpu_sc as plsc

HIDDEN = 128
B, T = 1024, 200
N_TOK = B * T
NUM_CORES = 2
NUM_WORKERS = 32
CHUNK = 128
PER_WORKER = N_TOK // NUM_WORKERS
CHUNKS_PER_WORKER = PER_WORKER // CHUNK
LANES = 16
VPR = HIDDEN // LANES


def _combo_body(pos_ref, type_ref, o_ref):
    pos = pos_ref[0:T, :]
    o_ref[0:T, :] = pos + type_ref[0, :][None, :]
    o_ref[T : 2 * T, :] = pos + type_ref[1, :][None, :]


def _combo_table(pos_w, type_w):
    return pl.pallas_call(
        _combo_body,
        out_shape=jax.ShapeDtypeStruct((2 * T, HIDDEN), jnp.float32),
    )(pos_w, type_w)


def _ln_rows(tok_v, cmb_v, r):
    x = [tok_v[r, pl.ds(j * LANES, LANES)] + cmb_v[r, pl.ds(j * LANES, LANES)]
         for j in range(VPR)]
    s = (((x[0] + x[1]) + (x[2] + x[3])) + ((x[4] + x[5]) + (x[6] + x[7])))
    q0 = [xi * xi for xi in x]
    q = (((q0[0] + q0[1]) + (q0[2] + q0[3])) + ((q0[4] + q0[5]) + (q0[6] + q0[7])))
    dn = lax.GatherDimensionNumbers(
        offset_dims=(), collapsed_slice_dims=(0,), start_index_map=(0,))
    lane = lax.iota(jnp.int32, LANES)
    def _lane_allsum(vec):
        for k in (1, 2, 4, 8):
            perm = (lane ^ k)[:, None]
            vec = vec + lax.gather(vec, perm, dn, slice_sizes=(1,),
                                   mode=lax.GatherScatterMode.PROMISE_IN_BOUNDS)
        return vec
    tot = _lane_allsum(s)
    qtot = _lane_allsum(q)
    mean_v = tot * (1.0 / HIDDEN)
    v = qtot * (1.0 / HIDDEN) - mean_v * mean_v + 1e-5
    i = lax.bitcast_convert_type(v, jnp.int32)
    y = lax.bitcast_convert_type(jnp.int32(0x5F3759DF) - (i >> 1), jnp.float32)
    for _ in range(3):
        y = y * (1.5 - 0.5 * v * y * y)
    for j in range(VPR):
        tok_v[r, pl.ds(j * LANES, LANES)] = (x[j] - mean_v) * y


def _sc_body(ids_hbm, cidx_hbm, tok_w_hbm, combo_hbm, out_hbm,
             idx_v, cidx_v, tok_v, cmb_v, sem):
    wid = lax.axis_index("s") * NUM_CORES + lax.axis_index("c")
    base = wid * PER_WORKER

    def chunk_body(i, carry):
        start = base + i * CHUNK
        pltpu.sync_copy(ids_hbm.at[pl.ds(start, CHUNK)], idx_v)
        pltpu.sync_copy(cidx_hbm.at[pl.ds(start, CHUNK)], cidx_v)
        pltpu.async_copy(tok_w_hbm.at[idx_v], tok_v, sem).wait()
        pltpu.async_copy(combo_hbm.at[cidx_v], cmb_v, sem).wait()

        def row_body(r, c):
            _ln_rows(tok_v, cmb_v, r)
            return c
        lax.fori_loop(0, CHUNK, row_body, 0)
        pltpu.sync_copy(tok_v, out_hbm.at[pl.ds(start, CHUNK)])
        return carry

    lax.fori_loop(0, CHUNKS_PER_WORKER, chunk_body, 0)


def _sc_fused(flat_ids, flat_cidx, token_w, combo):
    mesh = plsc.VectorSubcoreMesh(core_axis_name="c", subcore_axis_name="s")
    k = functools.partial(
        pl.kernel,
        mesh=mesh,
        out_type=jax.ShapeDtypeStruct((N_TOK, HIDDEN), jnp.float32),
        scratch_types=[
            pltpu.VMEM((CHUNK,), jnp.int32),
            pltpu.VMEM((CHUNK,), jnp.int32),
            pltpu.VMEM((CHUNK, HIDDEN), jnp.float32),
            pltpu.VMEM((CHUNK, HIDDEN), jnp.float32),
            pltpu.SemaphoreType.DMA,
        ],
    )(_sc_body)
    return k(flat_ids, flat_cidx, token_w, combo)


def kernel(input_ids, token_type_ids, token_w, pos_w, type_w, ln_w, ln_b):
    flat_ids = input_ids.reshape(-1)
    cidx = (token_type_ids * T + jnp.arange(T, dtype=jnp.int32)[None, :]).reshape(-1)
    combo = _combo_table(pos_w, type_w)
    out = _sc_fused(flat_ids, cidx, token_w, combo)
    return out.reshape(B, T, HIDDEN)


# speedup vs baseline: 1.1662x; 1.1662x over previous
import functools

import jax
import jax.numpy as jnp
from jax import lax
from jax.experimental import pallas as pl
from jax.experimental.pallas import tpu as pltpu
from jax.experimental.pallas import tpu_sc as plsc

HIDDEN = 128
B, T = 1024, 200
N_TOK = B * T
NUM_CORES = 2
NUM_WORKERS = 32
CHUNK = 128
PER_WORKER = N_TOK // NUM_WORKERS
CHUNKS_PER_WORKER = PER_WORKER // CHUNK
PAIRS = CHUNKS_PER_WORKER // 2
LANES = 16
VPR = HIDDEN // LANES

_DN = lax.GatherDimensionNumbers(
    offset_dims=(), collapsed_slice_dims=(0,), start_index_map=(0,))


def _lane_allsum(vec, lane):
    for k in (1, 2, 4, 8):
        perm = (lane ^ k)[:, None]
        vec = vec + lax.gather(vec, perm, _DN, slice_sizes=(1,),
                               mode=lax.GatherScatterMode.PROMISE_IN_BOUNDS)
    return vec


def _sc_body(ids_hbm, ttf_hbm, tok_w_hbm, pos_hbm, type_hbm, out_hbm,
             idx0, idx1, ttf0, ttf1, tok0, tok1, pos_v, type_v,
             st0, st1, so0, so1):
    wid = lax.axis_index("s") * NUM_CORES + lax.axis_index("c")
    base = wid * PER_WORKER
    idx_b = (idx0, idx1)
    ttf_b = (ttf0, ttf1)
    tok_b = (tok0, tok1)
    st_b = (st0, st1)

    pltpu.sync_copy(pos_hbm.at[pl.ds(0, T)], pos_v)
    pltpu.sync_copy(type_hbm, type_v)

    def fold_body(t, c):
        for j in range(VPR):
            sl = pl.ds(j * LANES, LANES)
            pos_v[t, sl] = pos_v[t, sl] + type_v[0, sl]
        return c
    lax.fori_loop(0, T, fold_body, 0, unroll=4)

    lane = lax.iota(jnp.int32, LANES)

    def stage(ii, b):
        start = base + ii * CHUNK
        pltpu.sync_copy(ids_hbm.at[pl.ds(start, CHUNK)], idx_b[b])
        pltpu.sync_copy(ttf_hbm.at[pl.ds(start, CHUNK)], ttf_b[b])
        pltpu.async_copy(tok_w_hbm.at[idx_b[b]], tok_b[b], st_b[b])

    def consume(ii, b):
        pltpu.make_async_copy(tok_w_hbm.at[idx_b[b]], tok_b[b], st_b[b]).wait()
        start = base + ii * CHUNK
        d = [type_v[1, pl.ds(j * LANES, LANES)] - type_v[0, pl.ds(j * LANES, LANES)]
             for j in range(VPR)]
        tokv = tok_b[b]
        ttfv = ttf_b[b]

        def group_body(g, c):
            ttg = ttfv[pl.ds(g * LANES, LANES)]
            for j in range(LANES):
                r = g * LANES + j
                t = lax.rem(start + r, T)
                splat_j = jnp.full((LANES, 1), j, dtype=jnp.int32)
                ttf = lax.gather(ttg, splat_j, _DN, slice_sizes=(1,),
                                 mode=lax.GatherScatterMode.PROMISE_IN_BOUNDS)
                x = [tokv[r, pl.ds(k * LANES, LANES)]
                     + pos_v[t, pl.ds(k * LANES, LANES)] + ttf * d[k]
                     for k in range(VPR)]
                s = ((x[0] + x[1]) + (x[2] + x[3])) + ((x[4] + x[5]) + (x[6] + x[7]))
                q0 = [xi * xi for xi in x]
                q = ((q0[0] + q0[1]) + (q0[2] + q0[3])) + ((q0[4] + q0[5]) + (q0[6] + q0[7]))
                tot = _lane_allsum(s, lane)
                qtot = _lane_allsum(q, lane)
                mean_v = tot * (1.0 / HIDDEN)
                v = qtot * (1.0 / HIDDEN) - mean_v * mean_v + 1e-5
                i = lax.bitcast_convert_type(v, jnp.int32)
                y = lax.bitcast_convert_type(jnp.int32(0x5F3759DF) - (i >> 1), jnp.float32)
                for _ in range(2):
                    y = y * (1.5 - 0.5 * v * y * y)
                for k in range(VPR):
                    tokv[r, pl.ds(k * LANES, LANES)] = (x[k] - mean_v) * y
            return c
        lax.fori_loop(0, CHUNK // LANES, group_body, 0)
        pltpu.sync_copy(tokv, out_hbm.at[pl.ds(start, CHUNK)])

    stage(0, 0)

    def pair_body(p, carry):
        e = 2 * p
        stage(e + 1, 1)
        consume(e, 0)

        @pl.when(p < PAIRS - 1)
        def _():
            stage(e + 2, 0)
        consume(e + 1, 1)
        return carry

    lax.fori_loop(0, PAIRS, pair_body, 0)


def _sc_fused(flat_ids, flat_ttf, token_w, pos_w, type_w):
    mesh = plsc.VectorSubcoreMesh(core_axis_name="c", subcore_axis_name="s")
    k = functools.partial(
        pl.kernel,
        mesh=mesh,
        out_type=jax.ShapeDtypeStruct((N_TOK, HIDDEN), jnp.float32),
        scratch_types=[
            pltpu.VMEM((CHUNK,), jnp.int32),
            pltpu.VMEM((CHUNK,), jnp.int32),
            pltpu.VMEM((CHUNK,), jnp.float32),
            pltpu.VMEM((CHUNK,), jnp.float32),
            pltpu.VMEM((CHUNK, HIDDEN), jnp.float32),
            pltpu.VMEM((CHUNK, HIDDEN), jnp.float32),
            pltpu.VMEM((T, HIDDEN), jnp.float32),
            pltpu.VMEM((2, HIDDEN), jnp.float32),
            pltpu.SemaphoreType.DMA,
            pltpu.SemaphoreType.DMA,
            pltpu.SemaphoreType.DMA,
            pltpu.SemaphoreType.DMA,
        ],
    )(_sc_body)
    return k(flat_ids, flat_ttf, token_w, pos_w, type_w)


def kernel(input_ids, token_type_ids, token_w, pos_w, type_w, ln_w, ln_b):
    flat_ids = input_ids.reshape(-1)
    flat_ttf = token_type_ids.astype(jnp.float32).reshape(-1)
    out = _sc_fused(flat_ids, flat_ttf, token_w, pos_w, type_w)
    return out.reshape(B, T, HIDDEN)


# speedup vs baseline: 1.7255x; 1.4796x over previous
import functools

import jax
import jax.numpy as jnp
from jax import lax
from jax.experimental import pallas as pl
from jax.experimental.pallas import tpu as pltpu
from jax.experimental.pallas import tpu_sc as plsc

VOCAB = 100000
HIDDEN = 128
B, T = 1024, 200
N_TOK = B * T
NUM_CORES = 2
NUM_WORKERS = 32
CHUNK = 128
PER_WORKER = N_TOK // NUM_WORKERS
CHUNKS_PER_WORKER = PER_WORKER // CHUNK
PAIRS = CHUNKS_PER_WORKER // 2


def _sc_body(ids_hbm, table_hbm, out_hbm, idx_all, rows0, rows1, s0, s1):
    wid = lax.axis_index("s") * NUM_CORES + lax.axis_index("c")
    base = wid * PER_WORKER
    rows_b = (rows0, rows1)
    sem_b = (s0, s1)

    pltpu.sync_copy(ids_hbm.at[pl.ds(base, PER_WORKER)], idx_all)

    def stage(ii, b):
        idx = idx_all.at[pl.ds(ii * CHUNK, CHUNK)]
        pltpu.async_copy(table_hbm.at[idx], rows_b[b], sem_b[b])

    def consume(ii, b):
        idx = idx_all.at[pl.ds(ii * CHUNK, CHUNK)]
        pltpu.make_async_copy(table_hbm.at[idx], rows_b[b], sem_b[b]).wait()
        pltpu.sync_copy(rows_b[b], out_hbm.at[pl.ds(base + ii * CHUNK, CHUNK)])

    stage(0, 0)

    def pair_body(p, carry):
        e = 2 * p
        stage(e + 1, 1)
        consume(e, 0)

        @pl.when(p < PAIRS - 1)
        def _():
            stage(e + 2, 0)
        consume(e + 1, 1)
        return carry

    lax.fori_loop(0, PAIRS, pair_body, 0)


def _sc_gather(flat_ids, token_w):
    mesh = plsc.VectorSubcoreMesh(core_axis_name="c", subcore_axis_name="s")
    k = functools.partial(
        pl.kernel,
        mesh=mesh,
        out_type=jax.ShapeDtypeStruct((N_TOK, HIDDEN), jnp.float32),
        scratch_types=[
            pltpu.VMEM((PER_WORKER,), jnp.int32),
            pltpu.VMEM((CHUNK, HIDDEN), jnp.float32),
            pltpu.VMEM((CHUNK, HIDDEN), jnp.float32),
            pltpu.SemaphoreType.DMA,
            pltpu.SemaphoreType.DMA,
        ],
    )(_sc_body)
    return k(flat_ids, token_w)


def _tc_ln_body(g_ref, ttf_ref, pos_ref, type_ref, lnw_ref, lnb_ref, o_ref):
    g = g_ref[...]
    ttf = ttf_ref[...]
    pos = pos_ref[...]
    t0 = type_ref[0, :][None, None, :]
    t1 = type_ref[1, :][None, None, :]
    te = t0 + ttf * (t1 - t0)
    x = g + pos[None] + te
    mean = jnp.mean(x, axis=-1, keepdims=True)
    xc = x - mean
    var = jnp.mean(xc * xc, axis=-1, keepdims=True)
    y = xc * lax.rsqrt(var + 1e-5)
    o_ref[...] = y * lnw_ref[...] + lnb_ref[...]


def _tc_ln(gathered, token_type_f, pos_w, type_w, ln_w, ln_b):
    BB = 8
    grid = (B // BB,)
    return pl.pallas_call(
        _tc_ln_body,
        grid=grid,
        in_specs=[
            pl.BlockSpec((BB, T, HIDDEN), lambda i: (i, 0, 0)),
            pl.BlockSpec((BB, T, 1), lambda i: (i, 0, 0)),
            pl.BlockSpec((T, HIDDEN), lambda i: (0, 0)),
            pl.BlockSpec((2, HIDDEN), lambda i: (0, 0)),
            pl.BlockSpec((HIDDEN,), lambda i: (0,)),
            pl.BlockSpec((HIDDEN,), lambda i: (0,)),
        ],
        out_specs=pl.BlockSpec((BB, T, HIDDEN), lambda i: (i, 0, 0)),
        out_shape=jax.ShapeDtypeStruct((B, T, HIDDEN), jnp.float32),
    )(gathered, token_type_f, pos_w, type_w, ln_w, ln_b)


def kernel(input_ids, token_type_ids, token_w, pos_w, type_w, ln_w, ln_b):
    flat_ids = input_ids.reshape(-1)
    gathered = _sc_gather(flat_ids, token_w).reshape(B, T, HIDDEN)
    ttf = token_type_ids.astype(jnp.float32).reshape(B, T, 1)
    return _tc_ln(gathered, ttf, pos_w, type_w, ln_w, ln_b)
